# Initial kernel scaffold; baseline (speedup 1.0000x reference)
import jax
import jax.numpy as jnp
from jax.experimental import pallas as pl


def kernel(node_types, node_values, depth, node_table, value_table, depth_table):
    raise NotImplementedError("write your pallas kernel here")


# speedup vs baseline: 4.1240x; 4.1240x over previous
import jax
import jax.numpy as jnp
from jax import lax
from jax.experimental import pallas as pl
from jax.experimental.pallas import tpu as pltpu
from jax.experimental.pallas import tpu_sc as plsc

HIDDEN_DIM = 128
MAX_DEPTH = 64
BATCH = 128
MAX_NODES = 256
VALUE_LEN = 32

NC, NS, L = 2, 16, 16
NW = NC * NS
TOTAL_NODES = BATCH * MAX_NODES
CHUNK = 16
NODES_PER_W = TOTAL_NODES // NW
CHUNKS_PER_W = NODES_PER_W // CHUNK
NUM_CHUNKS = TOTAL_NODES // CHUNK
VIDX_ROWS = CHUNK * VALUE_LEN // 128
COLS = HIDDEN_DIM // L


def _sc_body(nt_hbm, nv_hbm, dp_hbm, node_tab, val_tab, dep_tab, out_hbm,
             nidx, vidx, didx, vrows, nrows, drows, outv, sem):
  wid = lax.axis_index("s") * NC + lax.axis_index("c")

  def chunk_body(t, _):
    r = wid * CHUNKS_PER_W + t
    base = r * CHUNK

    pltpu.sync_copy(nt_hbm.at[r], nidx)
    pltpu.sync_copy(nv_hbm.at[r], vidx)
    pltpu.sync_copy(dp_hbm.at[r], didx)
    didx[...] = jnp.clip(didx[...], 0, MAX_DEPTH - 1)

    cps = []
    for k in range(VIDX_ROWS):
      cps.append(pltpu.async_copy(
          val_tab.at[vidx.at[k]], vrows.at[pl.ds(k * 128, 128)], sem))
    cps.append(pltpu.async_copy(node_tab.at[nidx], nrows, sem))
    cps.append(pltpu.async_copy(dep_tab.at[didx], drows, sem))
    for cp in cps:
      cp.wait()

    def node_body(i, _):
      vbase = i * VALUE_LEN
      for j in range(COLS):
        sl = pl.ds(j * L, L)
        acc = vrows[vbase, sl]
        for l in range(1, VALUE_LEN):
          acc = acc + vrows[vbase + l, sl]
        outv[i, sl] = nrows[i, sl] + drows[i, sl] + acc * (1.0 / VALUE_LEN)
      return 0

    lax.fori_loop(0, CHUNK, node_body, 0)
    pltpu.sync_copy(outv, out_hbm.at[pl.ds(base, CHUNK)])
    return 0

  lax.fori_loop(0, CHUNKS_PER_W, chunk_body, 0)


@jax.jit
def _tree_embed(nt, nv, dp, node_tab, val_tab, dep_tab):
  mesh = plsc.VectorSubcoreMesh(
      core_axis_name="c", subcore_axis_name="s", num_cores=NC, num_subcores=NS)
  return pl.kernel(
      _sc_body,
      out_type=jax.ShapeDtypeStruct((TOTAL_NODES, HIDDEN_DIM), jnp.float32),
      mesh=mesh,
      scratch_types=[
          pltpu.VMEM((CHUNK,), jnp.int32),
          pltpu.VMEM((VIDX_ROWS, 128), jnp.int32),
          pltpu.VMEM((CHUNK,), jnp.int32),
          pltpu.VMEM((CHUNK * VALUE_LEN, HIDDEN_DIM), jnp.float32),
          pltpu.VMEM((CHUNK, HIDDEN_DIM), jnp.float32),
          pltpu.VMEM((CHUNK, HIDDEN_DIM), jnp.float32),
          pltpu.VMEM((CHUNK, HIDDEN_DIM), jnp.float32),
          pltpu.SemaphoreType.DMA,
      ],
  )(nt, nv, dp, node_tab, val_tab, dep_tab)


def kernel(node_types, node_values, depth, node_table, value_table, depth_table):
  nt = node_types.reshape(NUM_CHUNKS, CHUNK).astype(jnp.int32)
  nv = node_values.reshape(NUM_CHUNKS, VIDX_ROWS, 128).astype(jnp.int32)
  dp = depth.reshape(NUM_CHUNKS, CHUNK).astype(jnp.int32)
  out = _tree_embed(nt, nv, dp, node_table, value_table, depth_table)
  return out.reshape(BATCH, MAX_NODES, HIDDEN_DIM)
